# baseline (speedup 1.0000x reference)
import functools

import jax
import jax.numpy as jnp
from jax import lax
from jax.experimental import pallas as pl
from jax.experimental.pallas import tpu as pltpu
from jax.experimental.pallas import tpu_sc as plsc

_BETA = 5.0
_H = 64
_NC = 2
_NS = 16
_NW = _NC * _NS
_CHUNK = 128
_LANES = 16


def _body(idx_hbm, emb_hbm, agency_hbm, boost_hbm, out_hbm,
          idx_v, g_v, e_v, b_v, sem_g, sem_b, sem_e):
    wid = lax.axis_index("s") * _NC + lax.axis_index("c")
    steps = idx_hbm.shape[1]

    pltpu.sync_copy(idx_hbm.at[wid], idx_v)

    def step(s, carry):
        cp_g = pltpu.async_copy(agency_hbm.at[idx_v.at[s]], g_v, sem_g)
        cp_b = pltpu.async_copy(boost_hbm.at[idx_v.at[s]], b_v, sem_b)
        cp_e = pltpu.async_copy(emb_hbm.at[wid, s], e_v, sem_e)
        cp_g.wait()
        cp_b.wait()
        cp_e.wait()

        def group(g, c):
            b16 = b_v[pl.ds(g * _LANES, _LANES)] * _BETA
            for t in range(_LANES):
                tok = g * _LANES + t
                coef = jnp.full((_LANES,), b16[t], jnp.float32)
                for j in range(_H // _LANES):
                    sl = pl.ds(j * _LANES, _LANES)
                    e_v[tok, sl] = e_v[tok, sl] + coef * g_v[tok, sl]
            return c

        lax.fori_loop(0, _CHUNK // _LANES, group, 0)
        pltpu.sync_copy(e_v, out_hbm.at[wid, s])
        return carry

    lax.fori_loop(0, steps, step, 0)


def kernel(embeddings, token_ids, agency_matrix, boosting_weights):
    B, L, H = embeddings.shape
    N = B * L
    steps = N // (_NW * _CHUNK)
    idx = token_ids.reshape(_NW, steps, _CHUNK).astype(jnp.int32)
    emb = embeddings.reshape(_NW, steps, _CHUNK, H)

    mesh = plsc.VectorSubcoreMesh(core_axis_name="c", subcore_axis_name="s")
    run = pl.kernel(
        _body,
        out_type=jax.ShapeDtypeStruct((_NW, steps, _CHUNK, H), jnp.float32),
        mesh=mesh,
        scratch_types=[
            pltpu.VMEM((steps, _CHUNK), jnp.int32),
            pltpu.VMEM((_CHUNK, H), jnp.float32),
            pltpu.VMEM((_CHUNK, H), jnp.float32),
            pltpu.VMEM((_CHUNK,), jnp.float32),
            pltpu.SemaphoreType.DMA,
            pltpu.SemaphoreType.DMA,
            pltpu.SemaphoreType.DMA,
        ],
        compiler_params=pltpu.CompilerParams(use_tc_tiling_on_sc=False),
    )
    out = run(idx, emb, agency_matrix, boosting_weights)
    return out.reshape(B, L, H)


# speedup vs baseline: 1.0747x; 1.0747x over previous
import functools

import jax
import jax.numpy as jnp
from jax import lax
from jax.experimental import pallas as pl
from jax.experimental.pallas import tpu as pltpu
from jax.experimental.pallas import tpu_sc as plsc

_BETA = 5.0
_H = 64
_NC = 2
_NS = 16
_NW = _NC * _NS
_CHUNK = 128
_LANES = 16


def _body(idx_hbm, emb_hbm, agency_hbm, boost_hbm, out_hbm,
          idx_v, g_v, e_v, o_v, b_v, sem_g, sem_b, sem_e, sem_o):
    wid = lax.axis_index("s") * _NC + lax.axis_index("c")
    steps = idx_hbm.shape[1]

    pltpu.sync_copy(idx_hbm.at[wid], idx_v)

    def gather_cp(s, slot):
        return pltpu.make_async_copy(
            agency_hbm.at[idx_v.at[s]], g_v.at[slot], sem_g.at[slot])

    def boost_cp(s, slot):
        return pltpu.make_async_copy(
            boost_hbm.at[idx_v.at[s]], b_v.at[slot], sem_b.at[slot])

    def emb_cp(s, slot):
        return pltpu.make_async_copy(
            emb_hbm.at[wid, s], e_v.at[slot], sem_e.at[slot])

    def out_cp(s, slot):
        return pltpu.make_async_copy(
            o_v.at[slot], out_hbm.at[wid, s], sem_o.at[slot])

    def start_in(s, slot):
        gather_cp(s, slot).start()
        boost_cp(s, slot).start()
        emb_cp(s, slot).start()

    def compute(slot):
        def group(g, c):
            b16 = b_v[slot, pl.ds(g * _LANES, _LANES)] * _BETA
            for t in range(_LANES):
                tok = g * _LANES + t
                coef = jnp.full((_LANES,), b16[t], jnp.float32)
                for j in range(_H // _LANES):
                    sl = pl.ds(j * _LANES, _LANES)
                    o_v[slot, tok, sl] = (e_v[slot, tok, sl]
                                          + coef * g_v[slot, tok, sl])
            return c

        lax.fori_loop(0, _CHUNK // _LANES, group, 0)

    n2 = steps // 2
    start_in(0, 0)

    def outer(i, c):
        for slot in range(2):
            s = i * 2 + slot
            if slot == 0:
                start_in(s + 1, 1)
            else:
                @pl.when(i < n2 - 1)
                def _():
                    start_in(s + 1, 0)
            gather_cp(s, slot).wait()
            boost_cp(s, slot).wait()
            emb_cp(s, slot).wait()

            @pl.when(i >= 1)
            def _():
                out_cp(s - 2, slot).wait()
            compute(slot)
            out_cp(s, slot).start()
        return c

    lax.fori_loop(0, n2, outer, 0)
    out_cp(steps - 2, 0).wait()
    out_cp(steps - 1, 1).wait()


def kernel(embeddings, token_ids, agency_matrix, boosting_weights):
    B, L, H = embeddings.shape
    N = B * L
    steps = N // (_NW * _CHUNK)
    idx = token_ids.reshape(_NW, steps, _CHUNK).astype(jnp.int32)
    emb = embeddings.reshape(_NW, steps, _CHUNK, H)

    mesh = plsc.VectorSubcoreMesh(core_axis_name="c", subcore_axis_name="s")
    run = pl.kernel(
        _body,
        out_type=jax.ShapeDtypeStruct((_NW, steps, _CHUNK, H), jnp.float32),
        mesh=mesh,
        scratch_types=[
            pltpu.VMEM((steps, _CHUNK), jnp.int32),
            pltpu.VMEM((2, _CHUNK, H), jnp.float32),
            pltpu.VMEM((2, _CHUNK, H), jnp.float32),
            pltpu.VMEM((2, _CHUNK, H), jnp.float32),
            pltpu.VMEM((2, _CHUNK), jnp.float32),
            pltpu.SemaphoreType.DMA((2,)),
            pltpu.SemaphoreType.DMA((2,)),
            pltpu.SemaphoreType.DMA((2,)),
            pltpu.SemaphoreType.DMA((2,)),
        ],
        compiler_params=pltpu.CompilerParams(use_tc_tiling_on_sc=False),
    )
    out = run(idx, emb, agency_matrix, boosting_weights)
    return out.reshape(B, L, H)


# speedup vs baseline: 1.0823x; 1.0071x over previous
import functools

import jax
import jax.numpy as jnp
from jax import lax
from jax.experimental import pallas as pl
from jax.experimental.pallas import tpu as pltpu
from jax.experimental.pallas import tpu_sc as plsc

_BETA = 5.0
_H = 64
_NC = 2
_NS = 16
_NW = _NC * _NS
_LANES = 16
_PARTS = ((0, 96), (96, 104))


def _body(idx_hbm, emb_hbm, agency_hbm, boost_hbm, out_hbm,
          idx_v, g_v, e_v, o_v, b_v, sem_g, sem_b, sem_e, sem_o):
    wid = lax.axis_index("s") * _NC + lax.axis_index("c")
    rows_per_w = idx_hbm.shape[0] // _NW
    L = idx_hbm.shape[1]
    base = wid * rows_per_w

    pltpu.sync_copy(idx_hbm.at[pl.ds(base, rows_per_w)], idx_v)

    def gather_cps(r, slot):
        return [pltpu.make_async_copy(
            agency_hbm.at[idx_v.at[r, pl.ds(off, ln)]],
            g_v.at[slot, pl.ds(off, ln)], sem_g.at[slot])
            for off, ln in _PARTS]

    def boost_cps(r, slot):
        return [pltpu.make_async_copy(
            boost_hbm.at[idx_v.at[r, pl.ds(off, ln)]],
            b_v.at[slot, pl.ds(off, ln)], sem_b.at[slot])
            for off, ln in _PARTS]

    def emb_cp(r, slot):
        return pltpu.make_async_copy(
            emb_hbm.at[base + r], e_v.at[slot], sem_e.at[slot])

    def out_cp(r, slot):
        return pltpu.make_async_copy(
            o_v.at[slot], out_hbm.at[base + r], sem_o.at[slot])

    def start_in(r, slot):
        for cp in gather_cps(r, slot):
            cp.start()
        for cp in boost_cps(r, slot):
            cp.start()
        emb_cp(r, slot).start()

    def wait_in(r, slot):
        for cp in gather_cps(r, slot):
            cp.wait()
        for cp in boost_cps(r, slot):
            cp.wait()
        emb_cp(r, slot).wait()

    def scale_add(slot, tok, coef):
        for j in range(_H // _LANES):
            sl = pl.ds(j * _LANES, _LANES)
            o_v[slot, tok, sl] = e_v[slot, tok, sl] + coef * g_v[slot, tok, sl]

    def compute(slot):
        def group(gi, c):
            b16 = b_v[slot, pl.ds(gi * _LANES, _LANES)] * _BETA
            for t in range(_LANES):
                scale_add(slot, gi * _LANES + t,
                          jnp.full((_LANES,), b16[t], jnp.float32))
            return c

        ngrp = L // _LANES
        lax.fori_loop(0, ngrp, group, 0)
        tail = L - ngrp * _LANES
        if tail:
            toff = L - _LANES
            b16 = b_v[slot, pl.ds(toff, _LANES)] * _BETA
            for t in range(_LANES - tail, _LANES):
                scale_add(slot, toff + t,
                          jnp.full((_LANES,), b16[t], jnp.float32))

    n2 = rows_per_w // 2
    start_in(0, 0)

    def outer(i, c):
        for slot in range(2):
            r = i * 2 + slot
            if slot == 0:
                start_in(r + 1, 1)
            else:
                @pl.when(i < n2 - 1)
                def _():
                    start_in(r + 1, 0)
            wait_in(r, slot)

            @pl.when(i >= 1)
            def _():
                out_cp(r - 2, slot).wait()
            compute(slot)
            out_cp(r, slot).start()
        return c

    lax.fori_loop(0, n2, outer, 0)
    out_cp(rows_per_w - 2, 0).wait()
    out_cp(rows_per_w - 1, 1).wait()


def kernel(embeddings, token_ids, agency_matrix, boosting_weights):
    B, L, H = embeddings.shape
    mesh = plsc.VectorSubcoreMesh(core_axis_name="c", subcore_axis_name="s")
    run = pl.kernel(
        _body,
        out_type=jax.ShapeDtypeStruct((B, L, H), jnp.float32),
        mesh=mesh,
        scratch_types=[
            pltpu.VMEM((B // _NW, L), jnp.int32),
            pltpu.VMEM((2, L, H), jnp.float32),
            pltpu.VMEM((2, L, H), jnp.float32),
            pltpu.VMEM((2, L, H), jnp.float32),
            pltpu.VMEM((2, L), jnp.float32),
            pltpu.SemaphoreType.DMA((2,)),
            pltpu.SemaphoreType.DMA((2,)),
            pltpu.SemaphoreType.DMA((2,)),
            pltpu.SemaphoreType.DMA((2,)),
        ],
        compiler_params=pltpu.CompilerParams(use_tc_tiling_on_sc=False),
    )
    return run(token_ids.astype(jnp.int32), embeddings,
               agency_matrix, boosting_weights)


# speedup vs baseline: 1.5480x; 1.4303x over previous
import functools

import jax
import jax.numpy as jnp
from jax import lax
from jax.experimental import pallas as pl
from jax.experimental.pallas import tpu as pltpu
from jax.experimental.pallas import tpu_sc as plsc

_BETA = 5.0
_H = 64
_HP = 128
_NC = 2
_NS = 16
_NW = _NC * _NS
_LANES = 16
_TBLK = 4096
_OFF = (0, 96)
_LEN = (96, 104)


def _transpose_body(at_ref, o_ref):
    o_ref[:, :_H] = at_ref[...].T


def _sc_body(tok_hbm, emb_hbm, table_hbm, boost_hbm, out_hbm, *s):
    idx = (s[0], s[1])
    g = (s[2], s[3])
    e = (s[4], s[5])
    o = (s[6], s[7])
    b = (s[8], s[9])
    sem_i = (s[10], s[11])
    sem_g = (s[12], s[13])
    sem_b = (s[14], s[15])
    sem_e = (s[16], s[17])
    sem_o = (s[18], s[19])

    wid = lax.axis_index("s") * _NC + lax.axis_index("c")
    L = emb_hbm.shape[1]
    rows_per_w = emb_hbm.shape[0] // _NW
    base = wid * rows_per_w

    def idx_cp(r, k):
        return pltpu.make_async_copy(
            tok_hbm.at[pl.ds((base + r) * L + _OFF[k], _LEN[k])],
            idx[k].at[pl.ds(0, _LEN[k])], sem_i[k])

    def gather_cp(k):
        return pltpu.make_async_copy(
            table_hbm.at[idx[k].at[pl.ds(0, _LEN[k])]],
            g[k].at[pl.ds(0, _LEN[k])], sem_g[k])

    def boost_cp(k):
        return pltpu.make_async_copy(
            boost_hbm.at[idx[k].at[pl.ds(0, _LEN[k])]],
            b[k].at[pl.ds(0, _LEN[k])], sem_b[k])

    def emb_cp(r, k):
        return pltpu.make_async_copy(
            emb_hbm.at[base + r, pl.ds(_OFF[k], _LEN[k])],
            e[k].at[pl.ds(0, _LEN[k])], sem_e[k])

    def out_cp(r, k):
        return pltpu.make_async_copy(
            o[k].at[pl.ds(0, _LEN[k])],
            out_hbm.at[base + r, pl.ds(_OFF[k], _LEN[k])], sem_o[k])

    def start_in(r, k):
        gather_cp(k).start()
        boost_cp(k).start()
        emb_cp(r, k).start()

    def wait_in(r, k):
        gather_cp(k).wait()
        boost_cp(k).wait()
        emb_cp(r, k).wait()

    def scale_add(k, tok, coef):
        for j in range(_H // _LANES):
            sl = pl.ds(j * _LANES, _LANES)
            o[k][tok, sl] = e[k][tok, sl] + coef * g[k][tok, sl]

    def compute(k):
        ln = _LEN[k]

        def group(gi, c):
            b16 = b[k][pl.ds(gi * _LANES, _LANES)] * _BETA
            for t in range(_LANES):
                scale_add(k, gi * _LANES + t,
                          jnp.full((_LANES,), b16[t], jnp.float32))
            return c

        ngrp = ln // _LANES
        lax.fori_loop(0, ngrp, group, 0)
        tail = ln - ngrp * _LANES
        if tail:
            toff = ln - _LANES
            b16 = b[k][pl.ds(toff, _LANES)] * _BETA
            for t in range(_LANES - tail, _LANES):
                scale_add(k, toff + t,
                          jnp.full((_LANES,), b16[t], jnp.float32))

    idx_cp(0, 0).start()
    idx_cp(0, 0).wait()
    start_in(0, 0)
    idx_cp(0, 1).start()

    def outer(r, c):
        for k in range(2):
            if k == 0:
                @pl.when(r < rows_per_w - 1)
                def _():
                    idx_cp(r + 1, 0).start()
                idx_cp(r, 1).wait()
                start_in(r, 1)
            else:
                @pl.when(r < rows_per_w - 1)
                def _():
                    idx_cp(r + 1, 1).start()
                    idx_cp(r + 1, 0).wait()
                    start_in(r + 1, 0)
            wait_in(r, k)

            @pl.when(r >= 1)
            def _():
                out_cp(r - 1, k).wait()
            compute(k)
            out_cp(r, k).start()
        return c

    lax.fori_loop(0, rows_per_w, outer, 0)
    out_cp(rows_per_w - 1, 0).wait()
    out_cp(rows_per_w - 1, 1).wait()


def kernel(embeddings, token_ids, agency_matrix, boosting_weights):
    B, L, H = embeddings.shape
    V = agency_matrix.shape[0]
    agency_t = agency_matrix.T

    table = pl.pallas_call(
        _transpose_body,
        grid=(pl.cdiv(V, _TBLK),),
        in_specs=[pl.BlockSpec((H, _TBLK), lambda i: (0, i))],
        out_specs=pl.BlockSpec((_TBLK, _HP), lambda i: (i, 0)),
        out_shape=jax.ShapeDtypeStruct((V, _HP), jnp.float32),
    )(agency_t)

    mesh = plsc.VectorSubcoreMesh(core_axis_name="c", subcore_axis_name="s")
    mx = _LEN[1]
    run = pl.kernel(
        _sc_body,
        out_type=jax.ShapeDtypeStruct((B, L, H), jnp.float32),
        mesh=mesh,
        scratch_types=(
            [pltpu.VMEM((mx,), jnp.int32) for _ in range(2)] +
            [pltpu.VMEM((mx, _HP), jnp.float32) for _ in range(2)] +
            [pltpu.VMEM((mx, _H), jnp.float32) for _ in range(2)] +
            [pltpu.VMEM((mx, _H), jnp.float32) for _ in range(2)] +
            [pltpu.VMEM((mx,), jnp.float32) for _ in range(2)] +
            [pltpu.SemaphoreType.DMA for _ in range(10)]
        ),
        compiler_params=pltpu.CompilerParams(use_tc_tiling_on_sc=True,
                                             needs_layout_passes=False),
    )
    return run(token_ids.astype(jnp.int32).reshape(B * L), embeddings, table,
               boosting_weights)


# speedup vs baseline: 1.7559x; 1.1343x over previous
import functools

import jax
import jax.numpy as jnp
from jax import lax
from jax.experimental import pallas as pl
from jax.experimental.pallas import tpu as pltpu
from jax.experimental.pallas import tpu_sc as plsc

_BETA = 5.0
_H = 64
_HP = 128
_NC = 2
_NS = 16
_NW = _NC * _NS
_LANES = 16
_TBLK = 8192
_OFF = (0, 96)
_LEN = (96, 104)


def _transpose_body(at_ref, o_ref):
    o_ref[:, :_H] = at_ref[...].T


def _sc_body(tok_hbm, table_hbm, boost_hbm, out_hbm, *s):
    idx = (s[0], s[1])
    g = (s[2], s[3])
    o = (s[4], s[5])
    b = (s[6], s[7])
    sem_i = (s[8], s[9])
    sem_g = (s[10], s[11])
    sem_b = (s[12], s[13])
    sem_o = (s[14], s[15])

    wid = lax.axis_index("s") * _NC + lax.axis_index("c")
    L = out_hbm.shape[1]
    rows_per_w = out_hbm.shape[0] // _NW
    base = wid * rows_per_w

    def idx_cp(r, k):
        return pltpu.make_async_copy(
            tok_hbm.at[pl.ds((base + r) * L + _OFF[k], _LEN[k])],
            idx[k].at[pl.ds(0, _LEN[k])], sem_i[k])

    def gather_cp(k):
        return pltpu.make_async_copy(
            table_hbm.at[idx[k].at[pl.ds(0, _LEN[k])]],
            g[k].at[pl.ds(0, _LEN[k])], sem_g[k])

    def boost_cp(k):
        return pltpu.make_async_copy(
            boost_hbm.at[idx[k].at[pl.ds(0, _LEN[k])]],
            b[k].at[pl.ds(0, _LEN[k])], sem_b[k])

    def out_cp(r, k):
        return pltpu.make_async_copy(
            o[k].at[pl.ds(0, _LEN[k])],
            out_hbm.at[base + r, pl.ds(_OFF[k], _LEN[k])], sem_o[k])

    def start_in(r, k):
        gather_cp(k).start()
        boost_cp(k).start()

    def wait_in(r, k):
        gather_cp(k).wait()
        boost_cp(k).wait()

    def scale_add(k, tok, coef):
        for j in range(_H // _LANES):
            sl = pl.ds(j * _LANES, _LANES)
            o[k][tok, sl] = coef * g[k][tok, sl]

    def compute(k):
        ln = _LEN[k]

        def group(gi, c):
            b16 = b[k][pl.ds(gi * _LANES, _LANES)] * _BETA
            for t in range(_LANES):
                scale_add(k, gi * _LANES + t,
                          jnp.full((_LANES,), b16[t], jnp.float32))
            return c

        ngrp = ln // _LANES
        lax.fori_loop(0, ngrp, group, 0)
        tail = ln - ngrp * _LANES
        if tail:
            toff = ln - _LANES
            b16 = b[k][pl.ds(toff, _LANES)] * _BETA
            for t in range(_LANES - tail, _LANES):
                scale_add(k, toff + t,
                          jnp.full((_LANES,), b16[t], jnp.float32))

    idx_cp(0, 0).start()
    idx_cp(0, 0).wait()
    start_in(0, 0)
    idx_cp(0, 1).start()

    def outer(r, c):
        for k in range(2):
            if k == 0:
                @pl.when(r < rows_per_w - 1)
                def _():
                    idx_cp(r + 1, 0).start()
                idx_cp(r, 1).wait()
                start_in(r, 1)
            else:
                @pl.when(r < rows_per_w - 1)
                def _():
                    idx_cp(r + 1, 1).start()
                    idx_cp(r + 1, 0).wait()
                    start_in(r + 1, 0)
            wait_in(r, k)

            @pl.when(r >= 1)
            def _():
                out_cp(r - 1, k).wait()
            compute(k)
            out_cp(r, k).start()
        return c

    lax.fori_loop(0, rows_per_w, outer, 0)
    out_cp(rows_per_w - 1, 0).wait()
    out_cp(rows_per_w - 1, 1).wait()


def kernel(embeddings, token_ids, agency_matrix, boosting_weights):
    B, L, H = embeddings.shape
    V = agency_matrix.shape[0]
    agency_t = agency_matrix.T

    table = pl.pallas_call(
        _transpose_body,
        grid=(pl.cdiv(V, _TBLK),),
        in_specs=[pl.BlockSpec((H, _TBLK), lambda i: (0, i))],
        out_specs=pl.BlockSpec((_TBLK, _HP), lambda i: (i, 0)),
        out_shape=jax.ShapeDtypeStruct((V, _HP), jnp.float32),
    )(agency_t)

    mesh = plsc.VectorSubcoreMesh(core_axis_name="c", subcore_axis_name="s")
    mx = _LEN[1]
    run = pl.kernel(
        _sc_body,
        out_type=jax.ShapeDtypeStruct((B, L, H), jnp.float32),
        mesh=mesh,
        scratch_types=(
            [pltpu.VMEM((mx,), jnp.int32) for _ in range(2)] +
            [pltpu.VMEM((mx, _HP), jnp.float32) for _ in range(2)] +
            [pltpu.VMEM((mx, _H), jnp.float32) for _ in range(2)] +
            [pltpu.VMEM((mx,), jnp.float32) for _ in range(2)] +
            [pltpu.SemaphoreType.DMA for _ in range(8)]
        ),
        compiler_params=pltpu.CompilerParams(use_tc_tiling_on_sc=True,
                                             needs_layout_passes=False),
    )
    boosted = run(token_ids.astype(jnp.int32).reshape(B * L), table,
                  boosting_weights)
    return embeddings + boosted
